# baseline (speedup 1.0000x reference)
import functools

import jax
import jax.numpy as jnp
from jax import lax
from jax.experimental import pallas as pl
from jax.experimental.pallas import tpu as pltpu
from jax.experimental.pallas import tpu_sc as plsc

_NUM_POS = 8192
_EMB = 1024
_SC_ROWS = 4096
_CHUNK = 32
_NBUF = 3
_TC_BLK = 512


@functools.cache
def _sc_copy_kernel():
    info = plsc.get_sparse_core_info()
    nc, ns = info.num_cores, info.num_subcores
    nw = nc * ns
    rows_per_w = _SC_ROWS // nw
    nchunks = rows_per_w // _CHUNK
    mesh = plsc.VectorSubcoreMesh(core_axis_name="c", subcore_axis_name="s")

    @functools.partial(
        pl.kernel,
        mesh=mesh,
        out_type=jax.ShapeDtypeStruct((_SC_ROWS, _EMB), jnp.float32),
        scratch_types=[
            pltpu.VMEM((_NBUF, _CHUNK, _EMB), jnp.float32),
            pltpu.SemaphoreType.DMA,
            pltpu.SemaphoreType.DMA,
            pltpu.SemaphoreType.DMA,
            pltpu.SemaphoreType.DMA,
            pltpu.SemaphoreType.DMA,
            pltpu.SemaphoreType.DMA,
        ],
    )
    def copy_k(table_hbm, out_hbm, buf, si0, si1, si2, so0, so1, so2):
        sin = (si0, si1, si2)
        sout = (so0, so1, so2)
        wid = lax.axis_index("s") * nc + lax.axis_index("c")
        base = wid * rows_per_w
        hin = [None] * nchunks
        hout = [None] * nchunks

        def start_in(i):
            b = i % _NBUF
            if i >= _NBUF:
                hout[i - _NBUF].wait()
            hin[i] = pltpu.async_copy(
                table_hbm.at[pl.ds(base + i * _CHUNK, _CHUNK)],
                buf.at[b], sin[b])

        start_in(0)
        for i in range(nchunks):
            if i + 1 < nchunks:
                start_in(i + 1)
            b = i % _NBUF
            hin[i].wait()
            hout[i] = pltpu.async_copy(
                buf.at[b], out_hbm.at[pl.ds(base + i * _CHUNK, _CHUNK)],
                sout[b])
        for i in range(max(nchunks - _NBUF, 0), nchunks):
            hout[i].wait()

    return copy_k


@functools.cache
def _tc_copy_kernel():
    tc_rows = _NUM_POS - _SC_ROWS
    nblk = tc_rows // _TC_BLK
    off = _SC_ROWS // _TC_BLK

    def body(w_ref, o_ref):
        o_ref[...] = w_ref[...]

    return pl.pallas_call(
        body,
        grid=(nblk,),
        in_specs=[pl.BlockSpec((_TC_BLK, _EMB), lambda i: (i + off, 0))],
        out_specs=pl.BlockSpec((_TC_BLK, _EMB), lambda i: (i, 0)),
        out_shape=jax.ShapeDtypeStruct((tc_rows, _EMB), jnp.float32),
    )


def kernel(x, pos_embed_weight):
    del x
    sc_out = _sc_copy_kernel()(pos_embed_weight)
    tc_out = _tc_copy_kernel()(pos_embed_weight)
    out = jnp.concatenate([sc_out, tc_out], axis=0)
    return out[None]
